# baseline (speedup 1.0000x reference)
import functools

import jax
import jax.numpy as jnp
from jax import lax
from jax.experimental import pallas as pl
from jax.experimental.pallas import tpu as pltpu
from jax.experimental.pallas import tpu_sc as plsc

TILE = 256


def _router_body(x_ref, wr_ref, idx_ref, g_ref, xbf_ref):
    x = x_ref[...]
    xbf_ref[...] = x.astype(jnp.bfloat16)
    wr = wr_ref[...]
    logits = lax.dot_general(x, wr, (((1,), (1,)), ((), ())),
                             preferred_element_type=jnp.float32)
    s = jax.nn.sigmoid(logits)
    n, e = s.shape
    col = lax.broadcasted_iota(jnp.int32, (n, e), 1)
    m1 = jnp.max(s, axis=1, keepdims=True)
    i1 = jnp.min(jnp.where(s == m1, col, e), axis=1, keepdims=True)
    s2 = jnp.where(col == i1, -jnp.inf, s)
    m2 = jnp.max(s2, axis=1, keepdims=True)
    i2 = jnp.min(jnp.where(s2 == m2, col, e), axis=1, keepdims=True)
    denom = m1 + m2
    safe = denom > 1e-9
    g1 = jnp.where(safe, m1 / (denom + 1e-9), 0.5)
    g2 = jnp.where(safe, m2 / (denom + 1e-9), 0.5)
    idx_ref[...] = jnp.concatenate([i1, i2], axis=1)
    g_ref[...] = jnp.concatenate([g1, g2], axis=1)


def _router(xf, Wr):
    n, d = xf.shape
    return pl.pallas_call(
        _router_body,
        out_shape=(jax.ShapeDtypeStruct((n, 2), jnp.int32),
                   jax.ShapeDtypeStruct((n, 2), jnp.float32),
                   jax.ShapeDtypeStruct((n, d), jnp.bfloat16)),
    )(xf, Wr)


def _shared_body(x_ref, w1_ref, w3_ref, w2_ref, o_ref):
    x = x_ref[...]
    a = lax.dot_general(x, w1_ref[...], (((1,), (1,)), ((), ())),
                        preferred_element_type=jnp.float32)
    b = lax.dot_general(x, w3_ref[...], (((1,), (1,)), ((), ())),
                        preferred_element_type=jnp.float32)
    h = (a * jax.nn.sigmoid(a) * b).astype(jnp.bfloat16)
    o_ref[...] = lax.dot_general(h, w2_ref[...], (((1,), (1,)), ((), ())),
                                 preferred_element_type=jnp.float32)


def _shared(xf, w1s, w3s, w2s):
    n, d = xf.shape
    hs = w1s.shape[0]
    bt = 256
    return pl.pallas_call(
        _shared_body,
        grid=(n // bt,),
        in_specs=[
            pl.BlockSpec((bt, d), lambda i: (i, 0)),
            pl.BlockSpec((hs, d), lambda i: (0, 0)),
            pl.BlockSpec((hs, d), lambda i: (0, 0)),
            pl.BlockSpec((d, hs), lambda i: (0, 0)),
        ],
        out_specs=pl.BlockSpec((bt, d), lambda i: (i, 0)),
        out_shape=jax.ShapeDtypeStruct((n, d), jnp.float32),
    )(xf, w1s, w3s, w2s)


def _ffn_body(te_ref, xs_ref, w1_ref, w2_ref, gs_ref, eo_ref):
    del te_ref
    xb = xs_ref[...].astype(jnp.bfloat16)
    h = lax.dot_general(xb, w1_ref[0], (((1,), (1,)), ((), ())),
                        preferred_element_type=jnp.float32)
    h = (0.5 * h * (1.0 + lax.erf(h * 0.7071067811865476))).astype(jnp.bfloat16)
    eo = lax.dot_general(h, w2_ref[0], (((1,), (1,)), ((), ())),
                         preferred_element_type=jnp.float32)
    eo_ref[...] = eo * gs_ref[...]


def _ffn(te, xs, W1, W2, gs2d):
    np_, d = xs.shape
    _, hr, _ = W1.shape
    nt = np_ // TILE
    grid_spec = pltpu.PrefetchScalarGridSpec(
        num_scalar_prefetch=1,
        grid=(nt,),
        in_specs=[
            pl.BlockSpec((TILE, d), lambda t, te_r: (t, 0)),
            pl.BlockSpec((1, hr, d), lambda t, te_r: (te_r[t], 0, 0)),
            pl.BlockSpec((1, d, hr), lambda t, te_r: (te_r[t], 0, 0)),
            pl.BlockSpec((TILE, 1), lambda t, te_r: (t, 0)),
        ],
        out_specs=pl.BlockSpec((TILE, d), lambda t, te_r: (t, 0)),
    )
    return pl.pallas_call(
        _ffn_body,
        grid_spec=grid_spec,
        out_shape=jax.ShapeDtypeStruct((np_, d), jnp.float32),
    )(te, xs, W1, W2, gs2d)


def _sc_gather(xf, st, np_):
    d = xf.shape[1]
    info = plsc.get_sparse_core_info()
    nw = info.num_cores * info.num_subcores
    rows_per = np_ // nw
    n_ch = 8
    nbuf = 4
    ch = rows_per // n_ch
    mesh = plsc.VectorSubcoreMesh(core_axis_name="c", subcore_axis_name="s")

    @functools.partial(
        pl.kernel, mesh=mesh,
        out_type=jax.ShapeDtypeStruct((np_, d), jnp.float32),
        scratch_types=[pltpu.VMEM((rows_per,), jnp.int32)]
                      + [pltpu.VMEM((ch, d), jnp.float32)] * nbuf
                      + [pltpu.SemaphoreType.DMA] * (2 * nbuf),
    )
    def k(x_hbm, st_hbm, out_hbm, idx_v, *bufs_sems):
        rows = bufs_sems[:nbuf]
        gsem = bufs_sems[nbuf:2 * nbuf]
        wsem = bufs_sems[2 * nbuf:]
        wid = lax.axis_index("s") * info.num_cores + lax.axis_index("c")
        base0 = wid * rows_per
        pltpu.sync_copy(st_hbm.at[pl.ds(base0, rows_per)], idx_v)
        gets = [None] * n_ch
        puts = [None] * n_ch

        def fire(c):
            b = c % nbuf
            gets[c] = pltpu.async_copy(
                x_hbm.at[idx_v.at[pl.ds(c * ch, ch)]], rows[b], gsem[b])

        for c in range(min(nbuf, n_ch)):
            fire(c)
        for c in range(n_ch):
            b = c % nbuf
            gets[c].wait()
            puts[c] = pltpu.async_copy(
                rows[b], out_hbm.at[pl.ds(base0 + c * ch, ch)], wsem[b])
            nxt = c + nbuf
            if nxt < n_ch:
                puts[c].wait()
                fire(nxt)
        for c in range(max(0, n_ch - nbuf), n_ch):
            puts[c].wait()

    return k(xf, st)


def _sc_combine(shared, eo, p0, p1):
    n, d = shared.shape
    info = plsc.get_sparse_core_info()
    nw = info.num_cores * info.num_subcores
    tok_per = n // nw
    ch = 16
    n_ch = tok_per // ch
    mesh = plsc.VectorSubcoreMesh(core_axis_name="c", subcore_axis_name="s")

    @functools.partial(
        pl.kernel, mesh=mesh,
        out_type=jax.ShapeDtypeStruct((n, d), jnp.float32),
        scratch_types=[pltpu.VMEM((ch,), jnp.int32),
                       pltpu.VMEM((ch,), jnp.int32),
                       pltpu.VMEM((ch, d), jnp.float32),
                       pltpu.VMEM((ch, d), jnp.float32),
                       pltpu.VMEM((ch, d), jnp.float32),
                       pltpu.SemaphoreType.DMA],
    )
    def k(sh_hbm, eo_hbm, p0_hbm, p1_hbm, out_hbm, i0_v, i1_v, sh_v, a_v, b_v,
          sem):
        wid = lax.axis_index("s") * info.num_cores + lax.axis_index("c")
        base0 = wid * tok_per
        for c in range(n_ch):
            base = base0 + c * ch
            pltpu.sync_copy(p0_hbm.at[pl.ds(base, ch)], i0_v)
            pltpu.sync_copy(p1_hbm.at[pl.ds(base, ch)], i1_v)
            pltpu.sync_copy(sh_hbm.at[pl.ds(base, ch)], sh_v)
            cp_a = pltpu.async_copy(eo_hbm.at[i0_v], a_v, sem)
            cp_b = pltpu.async_copy(eo_hbm.at[i1_v], b_v, sem)
            cp_a.wait()
            cp_b.wait()

            def row(i, carry):
                def chunk(j, carry2):
                    sl = pl.ds(j * 16, 16)
                    sh_v[i, sl] = sh_v[i, sl] + a_v[i, sl] + b_v[i, sl]
                    return carry2
                return lax.fori_loop(0, d // 16, chunk, carry)

            lax.fori_loop(0, ch, row, 0)
            pltpu.sync_copy(sh_v, out_hbm.at[pl.ds(base, ch)])

    return k(shared, eo, p0, p1)


def _slot_layout(idx2, g2, n, e):
    nt = (n * 2) // TILE + e
    np_ = nt * TILE
    sel = (jax.nn.one_hot(idx2[:, 0], e, dtype=jnp.int32)
           + jax.nn.one_hot(idx2[:, 1], e, dtype=jnp.int32))
    cnt = jnp.sum(sel, axis=0)
    rank = jnp.cumsum(sel, axis=0) - sel
    gpad = ((cnt + TILE - 1) // TILE) * TILE
    ends = jnp.cumsum(gpad)
    off = ends - gpad
    pos_ne = off[None, :] + rank
    pos2 = jnp.take_along_axis(pos_ne, idx2, axis=1)
    tok = jnp.arange(n, dtype=jnp.int32)
    st = (jnp.zeros((np_,), jnp.int32)
          .at[pos2[:, 0]].set(tok)
          .at[pos2[:, 1]].set(tok))
    gs = (jnp.zeros((np_,), jnp.float32)
          .at[pos2[:, 0]].set(g2[:, 0])
          .at[pos2[:, 1]].set(g2[:, 1]))
    tile_start = jnp.arange(nt, dtype=jnp.int32) * TILE
    te = jnp.searchsorted(ends, tile_start, side="right").astype(jnp.int32)
    te = jnp.minimum(te, e - 1)
    return st, gs, pos2, te, np_


def kernel(x, Wr, w1s, w3s, w2s, W1, W2):
    bq, tq, d = x.shape
    n = bq * tq
    e = Wr.shape[0]
    xf = x.reshape(n, d)

    idx2, g2, xbf = _router(xf, Wr)
    st, gs, pos2, te, np_ = _slot_layout(idx2, g2, n, e)

    xs = _sc_gather(xf, st, np_)
    shared = _shared(xbf, w1s.astype(jnp.bfloat16), w3s.astype(jnp.bfloat16),
                     w2s.astype(jnp.bfloat16))
    eo = _ffn(te, xs, W1.astype(jnp.bfloat16), W2.astype(jnp.bfloat16),
              gs[:, None])
    out = _sc_combine(shared, eo, pos2[:, 0], pos2[:, 1])
    return out.reshape(bq, tq, d)


# speedup vs baseline: 1.6447x; 1.6447x over previous
import functools

import jax
import jax.numpy as jnp
from jax import lax
from jax.experimental import pallas as pl
from jax.experimental.pallas import tpu as pltpu
from jax.experimental.pallas import tpu_sc as plsc

TILE = 256


def _router_body(x_ref, wr_ref, idx_ref, g_ref):
    x = x_ref[...]
    wr = wr_ref[...]
    logits = lax.dot_general(x, wr, (((1,), (1,)), ((), ())),
                             preferred_element_type=jnp.float32)
    s = jax.nn.sigmoid(logits)
    n, e = s.shape
    col = lax.broadcasted_iota(jnp.int32, (n, e), 1)
    m1 = jnp.max(s, axis=1, keepdims=True)
    i1 = jnp.min(jnp.where(s == m1, col, e), axis=1, keepdims=True)
    s2 = jnp.where(col == i1, -jnp.inf, s)
    m2 = jnp.max(s2, axis=1, keepdims=True)
    i2 = jnp.min(jnp.where(s2 == m2, col, e), axis=1, keepdims=True)
    denom = m1 + m2
    safe = denom > 1e-9
    g1 = jnp.where(safe, m1 / (denom + 1e-9), 0.5)
    g2 = jnp.where(safe, m2 / (denom + 1e-9), 0.5)
    idx_ref[...] = jnp.concatenate([i1, i2], axis=1)
    g_ref[...] = jnp.concatenate([g1, g2], axis=1)


def _router(xf, Wr):
    n, d = xf.shape
    return pl.pallas_call(
        _router_body,
        out_shape=(jax.ShapeDtypeStruct((n, 2), jnp.int32),
                   jax.ShapeDtypeStruct((n, 2), jnp.float32)),
    )(xf, Wr)


def _shared_body(x_ref, w1_ref, w3_ref, w2_ref, o_ref):
    x = x_ref[...]
    a = lax.dot_general(x, w1_ref[...], (((1,), (1,)), ((), ())),
                        preferred_element_type=jnp.float32)
    b = lax.dot_general(x, w3_ref[...], (((1,), (1,)), ((), ())),
                        preferred_element_type=jnp.float32)
    h = a * jax.nn.sigmoid(a) * b
    o_ref[...] = lax.dot_general(h, w2_ref[...], (((1,), (1,)), ((), ())),
                                 preferred_element_type=jnp.float32)


def _shared(xf, w1s, w3s, w2s):
    n, d = xf.shape
    hs = w1s.shape[0]
    bt = 256
    return pl.pallas_call(
        _shared_body,
        grid=(n // bt,),
        in_specs=[
            pl.BlockSpec((bt, d), lambda i: (i, 0)),
            pl.BlockSpec((hs, d), lambda i: (0, 0)),
            pl.BlockSpec((hs, d), lambda i: (0, 0)),
            pl.BlockSpec((d, hs), lambda i: (0, 0)),
        ],
        out_specs=pl.BlockSpec((bt, d), lambda i: (i, 0)),
        out_shape=jax.ShapeDtypeStruct((n, d), jnp.float32),
    )(xf, w1s, w3s, w2s)


def _ffn_body(te_ref, xs_ref, w1_ref, w2_ref, gs_ref, eo_ref):
    del te_ref
    xb = xs_ref[...]
    h = lax.dot_general(xb, w1_ref[0], (((1,), (1,)), ((), ())),
                        preferred_element_type=jnp.float32)
    h = 0.5 * h * (1.0 + lax.erf(h * 0.7071067811865476))
    eo = lax.dot_general(h, w2_ref[0], (((1,), (1,)), ((), ())),
                         preferred_element_type=jnp.float32)
    eo_ref[...] = eo * gs_ref[...]


def _ffn(te, xs, W1, W2, gs2d):
    np_, d = xs.shape
    _, hr, _ = W1.shape
    nt = np_ // TILE
    grid_spec = pltpu.PrefetchScalarGridSpec(
        num_scalar_prefetch=1,
        grid=(nt,),
        in_specs=[
            pl.BlockSpec((TILE, d), lambda t, te_r: (t, 0)),
            pl.BlockSpec((1, hr, d), lambda t, te_r: (te_r[t], 0, 0)),
            pl.BlockSpec((1, d, hr), lambda t, te_r: (te_r[t], 0, 0)),
            pl.BlockSpec((TILE, 1), lambda t, te_r: (t, 0)),
        ],
        out_specs=pl.BlockSpec((TILE, d), lambda t, te_r: (t, 0)),
    )
    return pl.pallas_call(
        _ffn_body,
        grid_spec=grid_spec,
        out_shape=jax.ShapeDtypeStruct((np_, d), jnp.float32),
    )(te, xs, W1, W2, gs2d)


def _sc_gather(xf, st, np_):
    d = xf.shape[1]
    info = plsc.get_sparse_core_info()
    nw = info.num_cores * info.num_subcores
    rows_per = np_ // nw
    n_ch = 8
    nbuf = 4
    ch = rows_per // n_ch
    mesh = plsc.VectorSubcoreMesh(core_axis_name="c", subcore_axis_name="s")

    @functools.partial(
        pl.kernel, mesh=mesh,
        out_type=jax.ShapeDtypeStruct((np_, d), jnp.float32),
        scratch_types=[pltpu.VMEM((rows_per,), jnp.int32)]
                      + [pltpu.VMEM((ch, d), jnp.float32)] * nbuf
                      + [pltpu.SemaphoreType.DMA] * (2 * nbuf),
    )
    def k(x_hbm, st_hbm, out_hbm, idx_v, *bufs_sems):
        rows = bufs_sems[:nbuf]
        gsem = bufs_sems[nbuf:2 * nbuf]
        wsem = bufs_sems[2 * nbuf:]
        wid = lax.axis_index("s") * info.num_cores + lax.axis_index("c")
        base0 = wid * rows_per
        pltpu.sync_copy(st_hbm.at[pl.ds(base0, rows_per)], idx_v)
        gets = [None] * n_ch
        puts = [None] * n_ch

        def fire(c):
            b = c % nbuf
            gets[c] = pltpu.async_copy(
                x_hbm.at[idx_v.at[pl.ds(c * ch, ch)]], rows[b], gsem[b])

        for c in range(min(nbuf, n_ch)):
            fire(c)
        for c in range(n_ch):
            b = c % nbuf
            gets[c].wait()
            puts[c] = pltpu.async_copy(
                rows[b], out_hbm.at[pl.ds(base0 + c * ch, ch)], wsem[b])
            nxt = c + nbuf
            if nxt < n_ch:
                puts[c].wait()
                fire(nxt)
        for c in range(max(0, n_ch - nbuf), n_ch):
            puts[c].wait()

    return k(xf, st)


def _sc_combine(shared, eo, p0, p1):
    n, d = shared.shape
    info = plsc.get_sparse_core_info()
    nw = info.num_cores * info.num_subcores
    tok_per = n // nw
    ch = 16
    n_ch = tok_per // ch
    mesh = plsc.VectorSubcoreMesh(core_axis_name="c", subcore_axis_name="s")

    @functools.partial(
        pl.kernel, mesh=mesh,
        out_type=jax.ShapeDtypeStruct((n, d), jnp.float32),
        scratch_types=[pltpu.VMEM((ch,), jnp.int32),
                       pltpu.VMEM((ch,), jnp.int32),
                       pltpu.VMEM((ch, d), jnp.float32),
                       pltpu.VMEM((ch, d), jnp.float32),
                       pltpu.VMEM((ch, d), jnp.float32),
                       pltpu.SemaphoreType.DMA],
    )
    def k(sh_hbm, eo_hbm, p0_hbm, p1_hbm, out_hbm, i0_v, i1_v, sh_v, a_v, b_v,
          sem):
        wid = lax.axis_index("s") * info.num_cores + lax.axis_index("c")
        base0 = wid * tok_per
        for c in range(n_ch):
            base = base0 + c * ch
            pltpu.sync_copy(p0_hbm.at[pl.ds(base, ch)], i0_v)
            pltpu.sync_copy(p1_hbm.at[pl.ds(base, ch)], i1_v)
            pltpu.sync_copy(sh_hbm.at[pl.ds(base, ch)], sh_v)
            cp_a = pltpu.async_copy(eo_hbm.at[i0_v], a_v, sem)
            cp_b = pltpu.async_copy(eo_hbm.at[i1_v], b_v, sem)
            cp_a.wait()
            cp_b.wait()

            def row(i, carry):
                def chunk(j, carry2):
                    sl = pl.ds(j * 16, 16)
                    sh_v[i, sl] = sh_v[i, sl] + a_v[i, sl] + b_v[i, sl]
                    return carry2
                return lax.fori_loop(0, d // 16, chunk, carry)

            lax.fori_loop(0, ch, row, 0)
            pltpu.sync_copy(sh_v, out_hbm.at[pl.ds(base, ch)])

    return k(shared, eo, p0, p1)


def _slot_layout(idx2, g2, n, e):
    nt = (n * 2) // TILE + e
    np_ = nt * TILE
    sel = (jax.nn.one_hot(idx2[:, 0], e, dtype=jnp.int32)
           + jax.nn.one_hot(idx2[:, 1], e, dtype=jnp.int32))
    cnt = jnp.sum(sel, axis=0)
    rank = jnp.cumsum(sel, axis=0) - sel
    gpad = ((cnt + TILE - 1) // TILE) * TILE
    ends = jnp.cumsum(gpad)
    off = ends - gpad
    pos_ne = off[None, :] + rank
    pos2 = jnp.take_along_axis(pos_ne, idx2, axis=1)
    tok = jnp.arange(n, dtype=jnp.int32)
    st = ((jnp.arange(np_, dtype=jnp.int32) * 97) % n)
    st = st.at[pos2[:, 0]].set(tok).at[pos2[:, 1]].set(tok)
    gs = (jnp.zeros((np_,), jnp.float32)
          .at[pos2[:, 0]].set(g2[:, 0])
          .at[pos2[:, 1]].set(g2[:, 1]))
    tile_start = jnp.arange(nt, dtype=jnp.int32) * TILE
    te = jnp.searchsorted(ends, tile_start, side="right").astype(jnp.int32)
    te = jnp.minimum(te, e - 1)
    return st, gs, pos2, te, np_


def kernel(x, Wr, w1s, w3s, w2s, W1, W2):
    bq, tq, d = x.shape
    n = bq * tq
    e = Wr.shape[0]
    xf = x.reshape(n, d)

    idx2, g2 = _router(xf, Wr)
    st, gs, pos2, te, np_ = _slot_layout(idx2, g2, n, e)

    xs = _sc_gather(xf, st, np_)
    shared = _shared(xf, w1s, w3s, w2s)
    eo = _ffn(te, xs, W1, W2, gs[:, None])
    out = _sc_combine(shared, eo, pos2[:, 0], pos2[:, 1])
    return out.reshape(bq, tq, d)


# speedup vs baseline: 1.8128x; 1.1022x over previous
import functools

import jax
import jax.numpy as jnp
from jax import lax
from jax.experimental import pallas as pl
from jax.experimental.pallas import tpu as pltpu
from jax.experimental.pallas import tpu_sc as plsc

TILE = 256


def _shared_router_body(x_ref, w1_ref, w3_ref, w2_ref, wr_ref,
                        o_ref, idx_ref, g_ref):
    x = x_ref[...]
    a = lax.dot_general(x, w1_ref[...], (((1,), (1,)), ((), ())),
                        preferred_element_type=jnp.float32)
    b = lax.dot_general(x, w3_ref[...], (((1,), (1,)), ((), ())),
                        preferred_element_type=jnp.float32)
    h = a * jax.nn.sigmoid(a) * b
    o_ref[...] = lax.dot_general(h, w2_ref[...], (((1,), (1,)), ((), ())),
                                 preferred_element_type=jnp.float32)

    logits = lax.dot_general(x, wr_ref[...], (((1,), (1,)), ((), ())),
                             preferred_element_type=jnp.float32)
    s = jax.nn.sigmoid(logits)
    n, e = s.shape
    col = lax.broadcasted_iota(jnp.int32, (n, e), 1)
    m1 = jnp.max(s, axis=1, keepdims=True)
    i1 = jnp.min(jnp.where(s == m1, col, e), axis=1, keepdims=True)
    s2 = jnp.where(col == i1, -jnp.inf, s)
    m2 = jnp.max(s2, axis=1, keepdims=True)
    i2 = jnp.min(jnp.where(s2 == m2, col, e), axis=1, keepdims=True)
    denom = m1 + m2
    safe = denom > 1e-9
    g1 = jnp.where(safe, m1 / (denom + 1e-9), 0.5)
    g2 = jnp.where(safe, m2 / (denom + 1e-9), 0.5)
    idx_ref[...] = jnp.concatenate([i1, i2], axis=1)
    g_ref[...] = jnp.concatenate([g1, g2], axis=1)


def _shared_router(xf, w1s, w3s, w2s, Wr):
    n, d = xf.shape
    hs = w1s.shape[0]
    e = Wr.shape[0]
    bt = 256
    return pl.pallas_call(
        _shared_router_body,
        grid=(n // bt,),
        in_specs=[
            pl.BlockSpec((bt, d), lambda i: (i, 0)),
            pl.BlockSpec((hs, d), lambda i: (0, 0)),
            pl.BlockSpec((hs, d), lambda i: (0, 0)),
            pl.BlockSpec((d, hs), lambda i: (0, 0)),
            pl.BlockSpec((e, d), lambda i: (0, 0)),
        ],
        out_specs=(pl.BlockSpec((bt, d), lambda i: (i, 0)),
                   pl.BlockSpec((bt, 2), lambda i: (i, 0)),
                   pl.BlockSpec((bt, 2), lambda i: (i, 0))),
        out_shape=(jax.ShapeDtypeStruct((n, d), jnp.float32),
                   jax.ShapeDtypeStruct((n, 2), jnp.int32),
                   jax.ShapeDtypeStruct((n, 2), jnp.float32)),
    )(xf, w1s, w3s, w2s, Wr)


def _ffn_body(te_ref, nu_ref, xs_ref, w1_ref, w2_ref, gs_ref, eo_ref):
    del te_ref
    t = pl.program_id(0)

    @pl.when(t < nu_ref[0])
    def _():
        xb = xs_ref[...]
        h = lax.dot_general(xb, w1_ref[0], (((1,), (1,)), ((), ())),
                            preferred_element_type=jnp.float32)
        h = 0.5 * h * (1.0 + lax.erf(h * 0.7071067811865476))
        eo = lax.dot_general(h, w2_ref[0], (((1,), (1,)), ((), ())),
                             preferred_element_type=jnp.float32)
        eo_ref[...] = eo * gs_ref[...]


def _ffn(te, nu, xs, W1, W2, gs2d):
    np_, d = xs.shape
    _, hr, _ = W1.shape
    nt = np_ // TILE
    grid_spec = pltpu.PrefetchScalarGridSpec(
        num_scalar_prefetch=2,
        grid=(nt,),
        in_specs=[
            pl.BlockSpec((TILE, d), lambda t, te_r, nu_r: (t, 0)),
            pl.BlockSpec((1, hr, d), lambda t, te_r, nu_r: (te_r[t], 0, 0)),
            pl.BlockSpec((1, d, hr), lambda t, te_r, nu_r: (te_r[t], 0, 0)),
            pl.BlockSpec((TILE, 1), lambda t, te_r, nu_r: (t, 0)),
        ],
        out_specs=pl.BlockSpec((TILE, d), lambda t, te_r, nu_r: (t, 0)),
    )
    return pl.pallas_call(
        _ffn_body,
        grid_spec=grid_spec,
        out_shape=jax.ShapeDtypeStruct((np_, d), jnp.float32),
    )(te, nu, xs, W1, W2, gs2d)


def _sc_gather(xf, st, np_):
    d = xf.shape[1]
    info = plsc.get_sparse_core_info()
    nw = info.num_cores * info.num_subcores
    rows_per = np_ // nw
    n_ch = 8
    nbuf = 4
    ch = rows_per // n_ch
    mesh = plsc.VectorSubcoreMesh(core_axis_name="c", subcore_axis_name="s")

    @functools.partial(
        pl.kernel, mesh=mesh,
        out_type=jax.ShapeDtypeStruct((np_, d), jnp.float32),
        scratch_types=[pltpu.VMEM((rows_per,), jnp.int32)]
                      + [pltpu.VMEM((ch, d), jnp.float32)] * nbuf
                      + [pltpu.SemaphoreType.DMA] * (2 * nbuf),
    )
    def k(x_hbm, st_hbm, out_hbm, idx_v, *bufs_sems):
        rows = bufs_sems[:nbuf]
        gsem = bufs_sems[nbuf:2 * nbuf]
        wsem = bufs_sems[2 * nbuf:]
        wid = lax.axis_index("s") * info.num_cores + lax.axis_index("c")
        base0 = wid * rows_per
        pltpu.sync_copy(st_hbm.at[pl.ds(base0, rows_per)], idx_v)
        gets = [None] * n_ch
        puts = [None] * n_ch

        def fire(c):
            b = c % nbuf
            gets[c] = pltpu.async_copy(
                x_hbm.at[idx_v.at[pl.ds(c * ch, ch)]], rows[b], gsem[b])

        for c in range(min(nbuf, n_ch)):
            fire(c)
        for c in range(n_ch):
            b = c % nbuf
            gets[c].wait()
            puts[c] = pltpu.async_copy(
                rows[b], out_hbm.at[pl.ds(base0 + c * ch, ch)], wsem[b])
            nxt = c + nbuf
            if nxt < n_ch:
                puts[c].wait()
                fire(nxt)
        for c in range(max(0, n_ch - nbuf), n_ch):
            puts[c].wait()

    return k(xf, st)


def _sc_combine(shared, eo, p0, p1):
    n, d = shared.shape
    info = plsc.get_sparse_core_info()
    nw = info.num_cores * info.num_subcores
    tok_per = n // nw
    ch = 16
    n_ch = tok_per // ch
    mesh = plsc.VectorSubcoreMesh(core_axis_name="c", subcore_axis_name="s")

    @functools.partial(
        pl.kernel, mesh=mesh,
        out_type=jax.ShapeDtypeStruct((n, d), jnp.float32),
        scratch_types=[pltpu.VMEM((tok_per,), jnp.int32),
                       pltpu.VMEM((tok_per,), jnp.int32)]
                      + [pltpu.VMEM((ch, d), jnp.float32)] * 6
                      + [pltpu.SemaphoreType.DMA] * 4,
    )
    def k(sh_hbm, eo_hbm, p0_hbm, p1_hbm, out_hbm, i0_v, i1_v, *bufs_sems):
        sh = bufs_sems[0:2]
        av = bufs_sems[2:4]
        bv = bufs_sems[4:6]
        gsem = bufs_sems[6:8]
        wsem = bufs_sems[8:10]
        wid = lax.axis_index("s") * info.num_cores + lax.axis_index("c")
        base0 = wid * tok_per
        pltpu.sync_copy(p0_hbm.at[pl.ds(base0, tok_per)], i0_v)
        pltpu.sync_copy(p1_hbm.at[pl.ds(base0, tok_per)], i1_v)
        gets = [None] * n_ch
        puts = [None] * n_ch

        def fire(c):
            s = c % 2
            gets[c] = (
                pltpu.async_copy(sh_hbm.at[pl.ds(base0 + c * ch, ch)],
                                 sh[s], gsem[s]),
                pltpu.async_copy(eo_hbm.at[i0_v.at[pl.ds(c * ch, ch)]],
                                 av[s], gsem[s]),
                pltpu.async_copy(eo_hbm.at[i1_v.at[pl.ds(c * ch, ch)]],
                                 bv[s], gsem[s]),
            )

        fire(0)
        for c in range(n_ch):
            s = c % 2
            if c + 1 < n_ch:
                if c >= 1:
                    puts[c - 1].wait()
                fire(c + 1)
            for cp in gets[c]:
                cp.wait()

            def row(i, carry):
                def chunk(j, carry2):
                    sl = pl.ds(j * 16, 16)
                    sh[s][i, sl] = sh[s][i, sl] + av[s][i, sl] + bv[s][i, sl]
                    return carry2
                return lax.fori_loop(0, d // 16, chunk, carry)

            lax.fori_loop(0, ch, row, 0)
            puts[c] = pltpu.async_copy(
                sh[s], out_hbm.at[pl.ds(base0 + c * ch, ch)], wsem[s])
        for c in range(max(0, n_ch - 2), n_ch):
            puts[c].wait()

    return k(shared, eo, p0, p1)


def _slot_layout(idx2, g2, n, e):
    nt = (n * 2) // TILE + e
    np_ = nt * TILE
    sel = (jax.nn.one_hot(idx2[:, 0], e, dtype=jnp.int32)
           + jax.nn.one_hot(idx2[:, 1], e, dtype=jnp.int32))
    cnt = jnp.sum(sel, axis=0)
    rank = jnp.cumsum(sel, axis=0) - sel
    gpad = ((cnt + TILE - 1) // TILE) * TILE
    ends = jnp.cumsum(gpad)
    off = ends - gpad
    pos_ne = off[None, :] + rank
    pos2 = jnp.take_along_axis(pos_ne, idx2, axis=1)
    tok = jnp.arange(n, dtype=jnp.int32)
    st = (jnp.arange(np_, dtype=jnp.int32) * 97) % n
    st = st.at[pos2[:, 0]].set(tok).at[pos2[:, 1]].set(tok)
    gs = (jnp.zeros((np_,), jnp.float32)
          .at[pos2[:, 0]].set(g2[:, 0])
          .at[pos2[:, 1]].set(g2[:, 1]))
    tile_start = jnp.arange(nt, dtype=jnp.int32) * TILE
    te = jnp.searchsorted(ends, tile_start, side="right").astype(jnp.int32)
    te = jnp.minimum(te, e - 1)
    nu = (ends[-1] // TILE).astype(jnp.int32).reshape(1)
    return st, gs, pos2, te, nu, np_


def kernel(x, Wr, w1s, w3s, w2s, W1, W2):
    bq, tq, d = x.shape
    n = bq * tq
    e = Wr.shape[0]
    xf = x.reshape(n, d)

    shared, idx2, g2 = _shared_router(xf, w1s, w3s, w2s, Wr)
    st, gs, pos2, te, nu, np_ = _slot_layout(idx2, g2, n, e)

    xs = _sc_gather(xf, st, np_)
    eo = _ffn(te, nu, xs, W1, W2, gs[:, None])
    out = _sc_combine(shared, eo, pos2[:, 0], pos2[:, 1])
    return out.reshape(bq, tq, d)


# speedup vs baseline: 1.9059x; 1.0513x over previous
import functools

import jax
import jax.numpy as jnp
from jax import lax
from jax.experimental import pallas as pl
from jax.experimental.pallas import tpu as pltpu
from jax.experimental.pallas import tpu_sc as plsc

TILE = 256


def _shared_router_body(x_ref, w1_ref, w3_ref, w2_ref, wr_ref,
                        o_ref, idx_ref, g_ref, rk_ref, cnt_ref, carry_ref):
    x = x_ref[...]
    a = lax.dot_general(x, w1_ref[...], (((1,), (1,)), ((), ())),
                        preferred_element_type=jnp.float32)
    b = lax.dot_general(x, w3_ref[...], (((1,), (1,)), ((), ())),
                        preferred_element_type=jnp.float32)
    h = a * jax.nn.sigmoid(a) * b
    o_ref[...] = lax.dot_general(h, w2_ref[...], (((1,), (1,)), ((), ())),
                                 preferred_element_type=jnp.float32)

    logits = lax.dot_general(x, wr_ref[...], (((1,), (1,)), ((), ())),
                             preferred_element_type=jnp.float32)
    s = jax.nn.sigmoid(logits)
    n, e = s.shape
    col = lax.broadcasted_iota(jnp.int32, (n, e), 1)
    m1 = jnp.max(s, axis=1, keepdims=True)
    i1 = jnp.min(jnp.where(s == m1, col, e), axis=1, keepdims=True)
    s2 = jnp.where(col == i1, -jnp.inf, s)
    m2 = jnp.max(s2, axis=1, keepdims=True)
    i2 = jnp.min(jnp.where(s2 == m2, col, e), axis=1, keepdims=True)
    denom = m1 + m2
    safe = denom > 1e-9
    g1 = jnp.where(safe, m1 / (denom + 1e-9), 0.5)
    g2 = jnp.where(safe, m2 / (denom + 1e-9), 0.5)
    idx_ref[...] = jnp.concatenate([i1, i2], axis=1)
    g_ref[...] = jnp.concatenate([g1, g2], axis=1)

    i = pl.program_id(0)

    @pl.when(i == 0)
    def _():
        carry_ref[...] = jnp.zeros_like(carry_ref)

    sel = (jnp.where(col == i1, 1.0, 0.0)
           + jnp.where(col == i2, 1.0, 0.0))
    r = lax.broadcasted_iota(jnp.int32, (n, n), 0)
    c = lax.broadcasted_iota(jnp.int32, (n, n), 1)
    ltri = jnp.where(r > c, 1.0, 0.0)
    local_ex = lax.dot_general(ltri, sel, (((1,), (0,)), ((), ())),
                               preferred_element_type=jnp.float32)
    carry = carry_ref[...]
    rank = local_ex + carry
    r1 = jnp.sum(jnp.where(col == i1, rank, 0.0), axis=1, keepdims=True)
    r2 = jnp.sum(jnp.where(col == i2, rank, 0.0), axis=1, keepdims=True)
    rk_ref[...] = jnp.concatenate([r1, r2], axis=1).astype(jnp.int32)
    new_carry = carry + jnp.sum(sel, axis=0, keepdims=True)
    carry_ref[...] = new_carry
    cnt_ref[...] = new_carry.astype(jnp.int32)


def _shared_router(xf, w1s, w3s, w2s, Wr):
    n, d = xf.shape
    hs = w1s.shape[0]
    e = Wr.shape[0]
    bt = 256
    return pl.pallas_call(
        _shared_router_body,
        grid=(n // bt,),
        in_specs=[
            pl.BlockSpec((bt, d), lambda i: (i, 0)),
            pl.BlockSpec((hs, d), lambda i: (0, 0)),
            pl.BlockSpec((hs, d), lambda i: (0, 0)),
            pl.BlockSpec((d, hs), lambda i: (0, 0)),
            pl.BlockSpec((e, d), lambda i: (0, 0)),
        ],
        out_specs=(pl.BlockSpec((bt, d), lambda i: (i, 0)),
                   pl.BlockSpec((bt, 2), lambda i: (i, 0)),
                   pl.BlockSpec((bt, 2), lambda i: (i, 0)),
                   pl.BlockSpec((bt, 2), lambda i: (i, 0)),
                   pl.BlockSpec((1, e), lambda i: (0, 0))),
        out_shape=(jax.ShapeDtypeStruct((n, d), jnp.float32),
                   jax.ShapeDtypeStruct((n, 2), jnp.int32),
                   jax.ShapeDtypeStruct((n, 2), jnp.float32),
                   jax.ShapeDtypeStruct((n, 2), jnp.int32),
                   jax.ShapeDtypeStruct((1, e), jnp.int32)),
        scratch_shapes=[pltpu.VMEM((1, e), jnp.float32)],
    )(xf, w1s, w3s, w2s, Wr)


def _layout_body(cnt_ref, idx_ref, rk_ref, pos_ref, te_ref, nu_ref):
    e = cnt_ref.shape[1]
    nt = te_ref.shape[0]
    cnt = cnt_ref[...].astype(jnp.float32)
    gpad = jnp.floor((cnt + (TILE - 1)) / TILE) * TILE
    ce = lax.broadcasted_iota(jnp.int32, (e, e), 0)
    re = lax.broadcasted_iota(jnp.int32, (e, e), 1)
    incl = jnp.where(ce <= re, 1.0, 0.0)
    ends = lax.dot_general(gpad, incl, (((1,), (0,)), ((), ())),
                           preferred_element_type=jnp.float32)
    off = ends - gpad
    n = idx_ref.shape[0]
    idx2 = idx_ref[...]
    rk = rk_ref[...].astype(jnp.float32)
    col = lax.broadcasted_iota(jnp.int32, (n, e), 1)
    p1 = jnp.sum(jnp.where(col == idx2[:, 0:1], off, 0.0), axis=1,
                 keepdims=True) + rk[:, 0:1]
    p2 = jnp.sum(jnp.where(col == idx2[:, 1:2], off, 0.0), axis=1,
                 keepdims=True) + rk[:, 1:2]
    pos_ref[...] = jnp.concatenate([p1, p2], axis=1).astype(jnp.int32)
    tstart = (lax.broadcasted_iota(jnp.int32, (nt, e), 0) * TILE
              ).astype(jnp.float32)
    ends_b = jnp.broadcast_to(ends, (nt, e))
    te = jnp.sum(jnp.where(ends_b <= tstart, 1, 0), axis=1, keepdims=True)
    te_ref[...] = jnp.minimum(te, e - 1).astype(jnp.int32)
    nu_ref[...] = (ends[:, e - 1:e] / TILE).astype(jnp.int32)


def _layout(cnt, idx2, rk, nt):
    n = idx2.shape[0]
    return pl.pallas_call(
        _layout_body,
        out_shape=(jax.ShapeDtypeStruct((n, 2), jnp.int32),
                   jax.ShapeDtypeStruct((nt, 1), jnp.int32),
                   jax.ShapeDtypeStruct((1, 1), jnp.int32)),
    )(cnt, idx2, rk)


def _ffn_body(te_ref, nu_ref, xs_ref, w1_ref, w2_ref, gs_ref, eo_ref):
    del te_ref
    t = pl.program_id(0)

    @pl.when(t < nu_ref[0])
    def _():
        xb = xs_ref[...]
        h = lax.dot_general(xb, w1_ref[0], (((1,), (1,)), ((), ())),
                            preferred_element_type=jnp.float32)
        h = 0.5 * h * (1.0 + lax.erf(h * 0.7071067811865476))
        eo = lax.dot_general(h, w2_ref[0], (((1,), (1,)), ((), ())),
                             preferred_element_type=jnp.float32)
        eo_ref[...] = eo * gs_ref[...]


def _ffn(te, nu, xs, W1, W2, gs2d):
    np_, d = xs.shape
    _, hr, _ = W1.shape
    nt = np_ // TILE
    grid_spec = pltpu.PrefetchScalarGridSpec(
        num_scalar_prefetch=2,
        grid=(nt,),
        in_specs=[
            pl.BlockSpec((TILE, d), lambda t, te_r, nu_r: (t, 0)),
            pl.BlockSpec((1, hr, d), lambda t, te_r, nu_r: (te_r[t], 0, 0)),
            pl.BlockSpec((1, d, hr), lambda t, te_r, nu_r: (te_r[t], 0, 0)),
            pl.BlockSpec((TILE, 1), lambda t, te_r, nu_r: (t, 0)),
        ],
        out_specs=pl.BlockSpec((TILE, d), lambda t, te_r, nu_r: (t, 0)),
    )
    return pl.pallas_call(
        _ffn_body,
        grid_spec=grid_spec,
        out_shape=jax.ShapeDtypeStruct((np_, d), jnp.float32),
    )(te, nu, xs, W1, W2, gs2d)


def _sc_gather(xf, st, np_):
    d = xf.shape[1]
    info = plsc.get_sparse_core_info()
    nw = info.num_cores * info.num_subcores
    rows_per = np_ // nw
    n_ch = 8
    nbuf = 4
    ch = rows_per // n_ch
    mesh = plsc.VectorSubcoreMesh(core_axis_name="c", subcore_axis_name="s")

    @functools.partial(
        pl.kernel, mesh=mesh,
        out_type=jax.ShapeDtypeStruct((np_, d), jnp.float32),
        scratch_types=[pltpu.VMEM((rows_per,), jnp.int32)]
                      + [pltpu.VMEM((ch, d), jnp.float32)] * nbuf
                      + [pltpu.SemaphoreType.DMA] * (2 * nbuf),
    )
    def k(x_hbm, st_hbm, out_hbm, idx_v, *bufs_sems):
        rows = bufs_sems[:nbuf]
        gsem = bufs_sems[nbuf:2 * nbuf]
        wsem = bufs_sems[2 * nbuf:]
        wid = lax.axis_index("s") * info.num_cores + lax.axis_index("c")
        base0 = wid * rows_per
        pltpu.sync_copy(st_hbm.at[pl.ds(base0, rows_per)], idx_v)
        gets = [None] * n_ch
        puts = [None] * n_ch

        def fire(c):
            b = c % nbuf
            gets[c] = pltpu.async_copy(
                x_hbm.at[idx_v.at[pl.ds(c * ch, ch)]], rows[b], gsem[b])

        for c in range(min(nbuf, n_ch)):
            fire(c)
        for c in range(n_ch):
            b = c % nbuf
            gets[c].wait()
            puts[c] = pltpu.async_copy(
                rows[b], out_hbm.at[pl.ds(base0 + c * ch, ch)], wsem[b])
            nxt = c + nbuf
            if nxt < n_ch:
                puts[c].wait()
                fire(nxt)
        for c in range(max(0, n_ch - nbuf), n_ch):
            puts[c].wait()

    return k(xf, st)


def _sc_combine(shared, eo, p0, p1):
    n, d = shared.shape
    info = plsc.get_sparse_core_info()
    nw = info.num_cores * info.num_subcores
    tok_per = n // nw
    ch = 16
    n_ch = tok_per // ch
    mesh = plsc.VectorSubcoreMesh(core_axis_name="c", subcore_axis_name="s")

    @functools.partial(
        pl.kernel, mesh=mesh,
        out_type=jax.ShapeDtypeStruct((n, d), jnp.float32),
        scratch_types=[pltpu.VMEM((tok_per,), jnp.int32),
                       pltpu.VMEM((tok_per,), jnp.int32)]
                      + [pltpu.VMEM((ch, d), jnp.float32)] * 6
                      + [pltpu.SemaphoreType.DMA] * 4,
    )
    def k(sh_hbm, eo_hbm, p0_hbm, p1_hbm, out_hbm, i0_v, i1_v, *bufs_sems):
        sh = bufs_sems[0:2]
        av = bufs_sems[2:4]
        bv = bufs_sems[4:6]
        gsem = bufs_sems[6:8]
        wsem = bufs_sems[8:10]
        wid = lax.axis_index("s") * info.num_cores + lax.axis_index("c")
        base0 = wid * tok_per
        pltpu.sync_copy(p0_hbm.at[pl.ds(base0, tok_per)], i0_v)
        pltpu.sync_copy(p1_hbm.at[pl.ds(base0, tok_per)], i1_v)
        gets = [None] * n_ch
        puts = [None] * n_ch

        def fire(c):
            s = c % 2
            gets[c] = (
                pltpu.async_copy(sh_hbm.at[pl.ds(base0 + c * ch, ch)],
                                 sh[s], gsem[s]),
                pltpu.async_copy(eo_hbm.at[i0_v.at[pl.ds(c * ch, ch)]],
                                 av[s], gsem[s]),
                pltpu.async_copy(eo_hbm.at[i1_v.at[pl.ds(c * ch, ch)]],
                                 bv[s], gsem[s]),
            )

        fire(0)
        for c in range(n_ch):
            s = c % 2
            if c + 1 < n_ch:
                if c >= 1:
                    puts[c - 1].wait()
                fire(c + 1)
            for cp in gets[c]:
                cp.wait()

            def row(i, carry):
                def chunk(j, carry2):
                    sl = pl.ds(j * 16, 16)
                    sh[s][i, sl] = sh[s][i, sl] + av[s][i, sl] + bv[s][i, sl]
                    return carry2
                return lax.fori_loop(0, d // 16, chunk, carry)

            lax.fori_loop(0, ch, row, 0)
            puts[c] = pltpu.async_copy(
                sh[s], out_hbm.at[pl.ds(base0 + c * ch, ch)], wsem[s])
        for c in range(max(0, n_ch - 2), n_ch):
            puts[c].wait()

    return k(shared, eo, p0, p1)


def kernel(x, Wr, w1s, w3s, w2s, W1, W2):
    bq, tq, d = x.shape
    n = bq * tq
    e = Wr.shape[0]
    xf = x.reshape(n, d)
    nt = (n * 2) // TILE + e
    np_ = nt * TILE

    shared, idx2, g2, rk, cnt = _shared_router(xf, w1s, w3s, w2s, Wr)
    pos2, te2, nu2 = _layout(cnt, idx2, rk, nt)

    pos_flat = pos2.reshape(-1)
    tok2 = jnp.arange(2 * n, dtype=jnp.int32) // 2
    st = ((jnp.arange(np_, dtype=jnp.int32) * 97) % n).at[pos_flat].set(tok2)
    gs = jnp.zeros((np_,), jnp.float32).at[pos_flat].set(g2.reshape(-1))

    xs = _sc_gather(xf, st, np_)
    eo = _ffn(te2.reshape(nt), nu2.reshape(1), xs, W1, W2, gs[:, None])
    out = _sc_combine(shared, eo, pos2[:, 0], pos2[:, 1])
    return out.reshape(bq, tq, d)


# speedup vs baseline: 2.0544x; 1.0779x over previous
import functools

import jax
import jax.numpy as jnp
from jax import lax
from jax.experimental import pallas as pl
from jax.experimental.pallas import tpu as pltpu
from jax.experimental.pallas import tpu_sc as plsc

TILE = 256


def _shared_router_body(x_ref, w1_ref, w3_ref, w2_ref, wr_ref,
                        o_ref, idx_ref, g_ref, rk_ref, cnt_ref, carry_ref):
    x = x_ref[...]
    a = lax.dot_general(x, w1_ref[...], (((1,), (1,)), ((), ())),
                        preferred_element_type=jnp.float32)
    b = lax.dot_general(x, w3_ref[...], (((1,), (1,)), ((), ())),
                        preferred_element_type=jnp.float32)
    h = a * jax.nn.sigmoid(a) * b
    o_ref[...] = lax.dot_general(h, w2_ref[...], (((1,), (1,)), ((), ())),
                                 preferred_element_type=jnp.float32)

    logits = lax.dot_general(x, wr_ref[...], (((1,), (1,)), ((), ())),
                             preferred_element_type=jnp.float32)
    s = jax.nn.sigmoid(logits)
    n, e = s.shape
    col = lax.broadcasted_iota(jnp.int32, (n, e), 1)
    m1 = jnp.max(s, axis=1, keepdims=True)
    i1 = jnp.min(jnp.where(s == m1, col, e), axis=1, keepdims=True)
    s2 = jnp.where(col == i1, -jnp.inf, s)
    m2 = jnp.max(s2, axis=1, keepdims=True)
    i2 = jnp.min(jnp.where(s2 == m2, col, e), axis=1, keepdims=True)
    denom = m1 + m2
    safe = denom > 1e-9
    g1 = jnp.where(safe, m1 / (denom + 1e-9), 0.5)
    g2 = jnp.where(safe, m2 / (denom + 1e-9), 0.5)
    idx_ref[...] = jnp.concatenate([i1, i2], axis=1)
    g_ref[...] = jnp.concatenate([g1, g2], axis=1)

    i = pl.program_id(0)

    @pl.when(i == 0)
    def _():
        carry_ref[...] = jnp.zeros_like(carry_ref)

    sel = (jnp.where(col == i1, 1.0, 0.0)
           + jnp.where(col == i2, 1.0, 0.0))
    r = lax.broadcasted_iota(jnp.int32, (n, n), 0)
    c = lax.broadcasted_iota(jnp.int32, (n, n), 1)
    ltri = jnp.where(r > c, 1.0, 0.0)
    local_ex = lax.dot_general(ltri, sel, (((1,), (0,)), ((), ())),
                               preferred_element_type=jnp.float32)
    carry = carry_ref[...]
    rank = local_ex + carry
    r1 = jnp.sum(jnp.where(col == i1, rank, 0.0), axis=1, keepdims=True)
    r2 = jnp.sum(jnp.where(col == i2, rank, 0.0), axis=1, keepdims=True)
    rk_ref[...] = jnp.concatenate([r1, r2], axis=1).astype(jnp.int32)
    new_carry = carry + jnp.sum(sel, axis=0, keepdims=True)
    carry_ref[...] = new_carry
    cnt_ref[...] = new_carry.astype(jnp.int32)


def _shared_router(xf, w1s, w3s, w2s, Wr):
    n, d = xf.shape
    hs = w1s.shape[0]
    e = Wr.shape[0]
    bt = 256
    return pl.pallas_call(
        _shared_router_body,
        grid=(n // bt,),
        in_specs=[
            pl.BlockSpec((bt, d), lambda i: (i, 0)),
            pl.BlockSpec((hs, d), lambda i: (0, 0)),
            pl.BlockSpec((hs, d), lambda i: (0, 0)),
            pl.BlockSpec((d, hs), lambda i: (0, 0)),
            pl.BlockSpec((e, d), lambda i: (0, 0)),
        ],
        out_specs=(pl.BlockSpec((bt, d), lambda i: (i, 0)),
                   pl.BlockSpec((bt, 2), lambda i: (i, 0)),
                   pl.BlockSpec((bt, 2), lambda i: (i, 0)),
                   pl.BlockSpec((bt, 2), lambda i: (i, 0)),
                   pl.BlockSpec((1, e), lambda i: (0, 0))),
        out_shape=(jax.ShapeDtypeStruct((n, d), jnp.float32),
                   jax.ShapeDtypeStruct((n, 2), jnp.int32),
                   jax.ShapeDtypeStruct((n, 2), jnp.float32),
                   jax.ShapeDtypeStruct((n, 2), jnp.int32),
                   jax.ShapeDtypeStruct((1, e), jnp.int32)),
        scratch_shapes=[pltpu.VMEM((1, e), jnp.float32)],
    )(xf, w1s, w3s, w2s, Wr)


def _layout_body(cnt_ref, idx_ref, rk_ref, pos_ref, te_ref, nu_ref):
    e = cnt_ref.shape[1]
    nt = te_ref.shape[0]
    cnt = cnt_ref[...].astype(jnp.float32)
    gpad = jnp.floor((cnt + (TILE - 1)) / TILE) * TILE
    ce = lax.broadcasted_iota(jnp.int32, (e, e), 0)
    re = lax.broadcasted_iota(jnp.int32, (e, e), 1)
    incl = jnp.where(ce <= re, 1.0, 0.0)
    ends = lax.dot_general(gpad, incl, (((1,), (0,)), ((), ())),
                           preferred_element_type=jnp.float32)
    off = ends - gpad
    n = idx_ref.shape[0]
    idx2 = idx_ref[...]
    rk = rk_ref[...].astype(jnp.float32)
    col = lax.broadcasted_iota(jnp.int32, (n, e), 1)
    p1 = jnp.sum(jnp.where(col == idx2[:, 0:1], off, 0.0), axis=1,
                 keepdims=True) + rk[:, 0:1]
    p2 = jnp.sum(jnp.where(col == idx2[:, 1:2], off, 0.0), axis=1,
                 keepdims=True) + rk[:, 1:2]
    pos_ref[...] = jnp.concatenate([p1, p2], axis=1).astype(jnp.int32)
    tstart = (lax.broadcasted_iota(jnp.int32, (nt, e), 0) * TILE
              ).astype(jnp.float32)
    ends_b = jnp.broadcast_to(ends, (nt, e))
    te = jnp.sum(jnp.where(ends_b <= tstart, 1, 0), axis=1, keepdims=True)
    te_ref[...] = jnp.minimum(te, e - 1).astype(jnp.int32)
    nu_ref[...] = (ends[:, e - 1:e] / TILE).astype(jnp.int32)


def _layout(cnt, idx2, rk, nt):
    n = idx2.shape[0]
    return pl.pallas_call(
        _layout_body,
        out_shape=(jax.ShapeDtypeStruct((n, 2), jnp.int32),
                   jax.ShapeDtypeStruct((nt, 1), jnp.int32),
                   jax.ShapeDtypeStruct((1, 1), jnp.int32)),
    )(cnt, idx2, rk)


def _ffn_body(te_ref, nu_ref, xs_ref, w1_ref, w2_ref, gs_ref, eo_ref):
    del te_ref
    t = pl.program_id(0)

    @pl.when(t < nu_ref[0])
    def _():
        xb = xs_ref[...]
        h = lax.dot_general(xb, w1_ref[0], (((1,), (1,)), ((), ())),
                            preferred_element_type=jnp.float32)
        h = 0.5 * h * (1.0 + lax.erf(h * 0.7071067811865476))
        eo = lax.dot_general(h, w2_ref[0], (((1,), (1,)), ((), ())),
                             preferred_element_type=jnp.float32)
        eo_ref[...] = eo * gs_ref[...]


def _ffn(te, nu, xs, W1, W2, gs2d):
    np_, d = xs.shape
    _, hr, _ = W1.shape
    nt = np_ // TILE
    grid_spec = pltpu.PrefetchScalarGridSpec(
        num_scalar_prefetch=2,
        grid=(nt,),
        in_specs=[
            pl.BlockSpec((TILE, d), lambda t, te_r, nu_r: (t, 0)),
            pl.BlockSpec((1, hr, d), lambda t, te_r, nu_r: (te_r[t], 0, 0)),
            pl.BlockSpec((1, d, hr), lambda t, te_r, nu_r: (te_r[t], 0, 0)),
            pl.BlockSpec((TILE, 1), lambda t, te_r, nu_r: (t, 0)),
        ],
        out_specs=pl.BlockSpec((TILE, d), lambda t, te_r, nu_r: (t, 0)),
    )
    return pl.pallas_call(
        _ffn_body,
        grid_spec=grid_spec,
        out_shape=jax.ShapeDtypeStruct((np_, d), jnp.float32),
    )(te, nu, xs, W1, W2, gs2d)


def _sc_gather(xf, st, np_):
    d = xf.shape[1]
    info = plsc.get_sparse_core_info()
    nw = info.num_cores * info.num_subcores
    rows_per = np_ // nw
    n_ch = 8
    nbuf = 4
    ch = rows_per // n_ch
    mesh = plsc.VectorSubcoreMesh(core_axis_name="c", subcore_axis_name="s")

    @functools.partial(
        pl.kernel, mesh=mesh,
        out_type=jax.ShapeDtypeStruct((np_, d), jnp.float32),
        scratch_types=[pltpu.VMEM((rows_per,), jnp.int32)]
                      + [pltpu.VMEM((ch, d), jnp.float32)] * nbuf
                      + [pltpu.SemaphoreType.DMA] * (2 * nbuf),
    )
    def k(x_hbm, st_hbm, out_hbm, idx_v, *bufs_sems):
        rows = bufs_sems[:nbuf]
        gsem = bufs_sems[nbuf:2 * nbuf]
        wsem = bufs_sems[2 * nbuf:]
        wid = lax.axis_index("s") * info.num_cores + lax.axis_index("c")
        base0 = wid * rows_per
        pltpu.sync_copy(st_hbm.at[pl.ds(base0, rows_per)], idx_v)
        gets = [None] * n_ch
        puts = [None] * n_ch

        def fire(c):
            b = c % nbuf
            gets[c] = pltpu.async_copy(
                x_hbm.at[idx_v.at[pl.ds(c * ch, ch)]], rows[b], gsem[b])

        for c in range(min(nbuf, n_ch)):
            fire(c)
        for c in range(n_ch):
            b = c % nbuf
            gets[c].wait()
            puts[c] = pltpu.async_copy(
                rows[b], out_hbm.at[pl.ds(base0 + c * ch, ch)], wsem[b])
            nxt = c + nbuf
            if nxt < n_ch:
                puts[c].wait()
                fire(nxt)
        for c in range(max(0, n_ch - nbuf), n_ch):
            puts[c].wait()

    return k(xf, st)


def _sc_combine(shared, eo, p0, p1):
    n, d = shared.shape
    info = plsc.get_sparse_core_info()
    nw = info.num_cores * info.num_subcores
    tok_per = n // nw
    ch = 16
    n_ch = tok_per // ch
    mesh = plsc.VectorSubcoreMesh(core_axis_name="c", subcore_axis_name="s")

    @functools.partial(
        pl.kernel, mesh=mesh,
        out_type=jax.ShapeDtypeStruct((n, d), jnp.float32),
        scratch_types=[pltpu.VMEM((tok_per,), jnp.int32),
                       pltpu.VMEM((tok_per,), jnp.int32)]
                      + [pltpu.VMEM((ch, d), jnp.float32)] * 6
                      + [pltpu.SemaphoreType.DMA] * 4,
    )
    def k(sh_hbm, eo_hbm, p0_hbm, p1_hbm, out_hbm, i0_v, i1_v, *bufs_sems):
        sh = bufs_sems[0:2]
        av = bufs_sems[2:4]
        bv = bufs_sems[4:6]
        gsem = bufs_sems[6:8]
        wsem = bufs_sems[8:10]
        wid = lax.axis_index("s") * info.num_cores + lax.axis_index("c")
        base0 = wid * tok_per
        pltpu.sync_copy(p0_hbm.at[pl.ds(base0, tok_per)], i0_v)
        pltpu.sync_copy(p1_hbm.at[pl.ds(base0, tok_per)], i1_v)
        gets = [None] * n_ch
        puts = [None] * n_ch

        def fire(c):
            s = c % 2
            gets[c] = (
                pltpu.async_copy(sh_hbm.at[pl.ds(base0 + c * ch, ch)],
                                 sh[s], gsem[s]),
                pltpu.async_copy(eo_hbm.at[i0_v.at[pl.ds(c * ch, ch)]],
                                 av[s], gsem[s]),
                pltpu.async_copy(eo_hbm.at[i1_v.at[pl.ds(c * ch, ch)]],
                                 bv[s], gsem[s]),
            )

        fire(0)
        for c in range(n_ch):
            s = c % 2
            if c + 1 < n_ch:
                if c >= 1:
                    puts[c - 1].wait()
                fire(c + 1)
            for cp in gets[c]:
                cp.wait()

            def row(i, carry):
                def chunk(j, carry2):
                    for u in range(4):
                        sl = pl.ds(j * 64 + u * 16, 16)
                        sh[s][i, sl] = (sh[s][i, sl] + av[s][i, sl]
                                        + bv[s][i, sl])
                    return carry2
                return lax.fori_loop(0, d // 64, chunk, carry)

            lax.fori_loop(0, ch, row, 0)
            puts[c] = pltpu.async_copy(
                sh[s], out_hbm.at[pl.ds(base0 + c * ch, ch)], wsem[s])
        for c in range(max(0, n_ch - 2), n_ch):
            puts[c].wait()

    return k(shared, eo, p0, p1)


def kernel(x, Wr, w1s, w3s, w2s, W1, W2):
    bq, tq, d = x.shape
    n = bq * tq
    e = Wr.shape[0]
    xf = x.reshape(n, d)
    nt = (n * 2) // TILE + e
    np_ = nt * TILE

    shared, idx2, g2, rk, cnt = _shared_router(xf, w1s, w3s, w2s, Wr)
    pos2, te2, nu2 = _layout(cnt, idx2, rk, nt)

    pos_flat = pos2.reshape(-1)
    tok2 = jnp.arange(2 * n, dtype=jnp.int32) // 2
    st = ((jnp.arange(np_, dtype=jnp.int32) * 97) % n).at[pos_flat].set(tok2)
    gs = jnp.zeros((np_,), jnp.float32).at[pos_flat].set(g2.reshape(-1))

    xs = _sc_gather(xf, st, np_)
    eo = _ffn(te2.reshape(nt), nu2.reshape(1), xs, W1, W2, gs[:, None])
    out = _sc_combine(shared, eo, pos2[:, 0], pos2[:, 1])
    return out.reshape(bq, tq, d)
